# baseline (speedup 1.0000x reference)
import jax
import jax.numpy as jnp
from jax.experimental import pallas as pl
from jax.experimental.pallas import tpu as pltpu

TM = 1024
TF = 512


def _mlp_kernel(col_ref, x_ref, w1_ref, b1_ref, w2_ref, b2_ref, o_ref, acc_ref):
    f = pl.program_id(1)
    nf = pl.num_programs(1)

    x = x_ref[...].astype(jnp.bfloat16)
    w1 = w1_ref[0].astype(jnp.bfloat16)
    h = jnp.dot(x, w1, preferred_element_type=jnp.float32)
    h = jax.nn.gelu(h + b1_ref[0])
    w2 = w2_ref[0].astype(jnp.bfloat16)
    p = jnp.dot(h.astype(jnp.bfloat16), w2, preferred_element_type=jnp.float32)

    @pl.when(f == 0)
    def _():
        acc_ref[...] = p

    @pl.when(f > 0)
    def _():
        acc_ref[...] += p

    @pl.when(f == nf - 1)
    def _():
        o_ref[...] = acc_ref[...] + b2_ref[0]


def kernel(hidden_states, W1, b1, W2, b2, col):
    T, D_MODEL = hidden_states.shape
    E, _, D_FF = W1.shape
    col_arr = jnp.asarray(col, jnp.int32).reshape((1,))
    b1r = b1.reshape(E, 1, D_FF)
    b2r = b2.reshape(E, 1, D_MODEL)

    grid = (T // TM, D_FF // TF)
    grid_spec = pltpu.PrefetchScalarGridSpec(
        num_scalar_prefetch=1,
        grid=grid,
        in_specs=[
            pl.BlockSpec((TM, D_MODEL), lambda m, f, c: (m, 0)),
            pl.BlockSpec((1, D_MODEL, TF), lambda m, f, c: (c[0], 0, f)),
            pl.BlockSpec((1, 1, TF), lambda m, f, c: (c[0], 0, f)),
            pl.BlockSpec((1, TF, D_MODEL), lambda m, f, c: (c[0], f, 0)),
            pl.BlockSpec((1, 1, D_MODEL), lambda m, f, c: (c[0], 0, 0)),
        ],
        out_specs=pl.BlockSpec((TM, D_MODEL), lambda m, f, c: (m, 0)),
        scratch_shapes=[pltpu.VMEM((TM, D_MODEL), jnp.float32)],
    )
    return pl.pallas_call(
        _mlp_kernel,
        grid_spec=grid_spec,
        out_shape=jax.ShapeDtypeStruct((T, D_MODEL), jnp.float32),
        compiler_params=pltpu.CompilerParams(
            dimension_semantics=("arbitrary", "arbitrary"),
        ),
    )(col_arr, hidden_states, W1, b1r, W2, b2r)


# speedup vs baseline: 1.0671x; 1.0671x over previous
import jax
import jax.numpy as jnp
from jax.experimental import pallas as pl
from jax.experimental.pallas import tpu as pltpu

TM = 2048
TF = 512


def _mlp_kernel(col_ref, x_ref, w1_ref, b1_ref, w2_ref, b2_ref, o_ref):
    f = pl.program_id(1)

    x = x_ref[...].astype(jnp.bfloat16)
    w1 = w1_ref[0].astype(jnp.bfloat16)
    h = jnp.dot(x, w1, preferred_element_type=jnp.float32)
    h = jax.nn.gelu(h + b1_ref[0])
    w2 = w2_ref[0].astype(jnp.bfloat16)
    p = jnp.dot(h.astype(jnp.bfloat16), w2, preferred_element_type=jnp.float32)

    @pl.when(f == 0)
    def _():
        o_ref[...] = p + b2_ref[0]

    @pl.when(f > 0)
    def _():
        o_ref[...] += p


def kernel(hidden_states, W1, b1, W2, b2, col):
    T, D_MODEL = hidden_states.shape
    E, _, D_FF = W1.shape
    col_arr = jnp.asarray(col, jnp.int32).reshape((1,))
    b1r = b1.reshape(E, 1, D_FF)
    b2r = b2.reshape(E, 1, D_MODEL)

    grid = (T // TM, D_FF // TF)
    grid_spec = pltpu.PrefetchScalarGridSpec(
        num_scalar_prefetch=1,
        grid=grid,
        in_specs=[
            pl.BlockSpec((TM, D_MODEL), lambda m, f, c: (m, 0)),
            pl.BlockSpec((1, D_MODEL, TF), lambda m, f, c: (c[0], 0, f)),
            pl.BlockSpec((1, 1, TF), lambda m, f, c: (c[0], 0, f)),
            pl.BlockSpec((1, TF, D_MODEL), lambda m, f, c: (c[0], f, 0)),
            pl.BlockSpec((1, 1, D_MODEL), lambda m, f, c: (c[0], 0, 0)),
        ],
        out_specs=pl.BlockSpec((TM, D_MODEL), lambda m, f, c: (m, 0)),
    )
    return pl.pallas_call(
        _mlp_kernel,
        grid_spec=grid_spec,
        out_shape=jax.ShapeDtypeStruct((T, D_MODEL), jnp.float32),
        compiler_params=pltpu.CompilerParams(
            dimension_semantics=("arbitrary", "arbitrary"),
        ),
    )(col_arr, hidden_states, W1, b1r, W2, b2r)


# speedup vs baseline: 1.1942x; 1.1192x over previous
import jax
import jax.numpy as jnp
from jax.experimental import pallas as pl
from jax.experimental.pallas import tpu as pltpu

TM = 512
TFP = 512


def _gather_cast_kernel(col_ref, w1_ref, b1_ref, w2_ref, b2_ref,
                        w1_out, b1_out, w2_out, b2_out):
    f = pl.program_id(0)
    w1_out[...] = w1_ref[0].astype(jnp.bfloat16)
    w2_out[...] = w2_ref[0].astype(jnp.bfloat16)
    b1_out[...] = b1_ref[0]

    @pl.when(f == 0)
    def _():
        b2_out[...] = b2_ref[0]


def _mlp_kernel(x_ref, w1_ref, b1_ref, w2_ref, b2_ref, o_ref):
    xb = x_ref[...].astype(jnp.bfloat16)
    h = jnp.dot(xb, w1_ref[...], preferred_element_type=jnp.float32)
    h = jax.nn.gelu(h + b1_ref[...])
    o_ref[...] = jnp.dot(h.astype(jnp.bfloat16), w2_ref[...],
                         preferred_element_type=jnp.float32) + b2_ref[...]


def kernel(hidden_states, W1, b1, W2, b2, col):
    T, D_MODEL = hidden_states.shape
    E, _, D_FF = W1.shape
    col_arr = jnp.asarray(col, jnp.int32).reshape((1,))
    b1r = b1.reshape(E, 1, D_FF)
    b2r = b2.reshape(E, 1, D_MODEL)

    prep_spec = pltpu.PrefetchScalarGridSpec(
        num_scalar_prefetch=1,
        grid=(D_FF // TFP,),
        in_specs=[
            pl.BlockSpec((1, D_MODEL, TFP), lambda f, c: (c[0], 0, f)),
            pl.BlockSpec((1, 1, TFP), lambda f, c: (c[0], 0, f)),
            pl.BlockSpec((1, TFP, D_MODEL), lambda f, c: (c[0], f, 0)),
            pl.BlockSpec((1, 1, D_MODEL), lambda f, c: (c[0], 0, 0)),
        ],
        out_specs=[
            pl.BlockSpec((D_MODEL, TFP), lambda f, c: (0, f)),
            pl.BlockSpec((1, TFP), lambda f, c: (0, f)),
            pl.BlockSpec((TFP, D_MODEL), lambda f, c: (f, 0)),
            pl.BlockSpec((1, D_MODEL), lambda f, c: (0, 0)),
        ],
    )
    w1b, b1g, w2b, b2g = pl.pallas_call(
        _gather_cast_kernel,
        grid_spec=prep_spec,
        out_shape=[
            jax.ShapeDtypeStruct((D_MODEL, D_FF), jnp.bfloat16),
            jax.ShapeDtypeStruct((1, D_FF), jnp.float32),
            jax.ShapeDtypeStruct((D_FF, D_MODEL), jnp.bfloat16),
            jax.ShapeDtypeStruct((1, D_MODEL), jnp.float32),
        ],
    )(col_arr, W1, b1r, W2, b2r)

    return pl.pallas_call(
        _mlp_kernel,
        grid=(T // TM,),
        in_specs=[
            pl.BlockSpec((TM, D_MODEL), lambda m: (m, 0)),
            pl.BlockSpec((D_MODEL, D_FF), lambda m: (0, 0)),
            pl.BlockSpec((1, D_FF), lambda m: (0, 0)),
            pl.BlockSpec((D_FF, D_MODEL), lambda m: (0, 0)),
            pl.BlockSpec((1, D_MODEL), lambda m: (0, 0)),
        ],
        out_specs=pl.BlockSpec((TM, D_MODEL), lambda m: (m, 0)),
        out_shape=jax.ShapeDtypeStruct((T, D_MODEL), jnp.float32),
        compiler_params=pltpu.CompilerParams(
            dimension_semantics=("arbitrary",),
        ),
    )(hidden_states, w1b, b1g, w2b, b2g)


# speedup vs baseline: 1.2509x; 1.0475x over previous
import jax
import jax.numpy as jnp
from jax.experimental import pallas as pl
from jax.experimental.pallas import tpu as pltpu

TM = 512
NB = 4
CB = 1024


def _half_gelu(h):
    c0 = 0.7978845608028654
    c1 = 0.044715 * c0
    u = h * (c0 + (h * h) * c1)
    t = jnp.tanh(u)
    return h + h * t


def _mlp_kernel(col_ref, x_ref, w1_ref, b1_ref, w2_ref, b2_ref, o_ref,
                w1s, w2s, b1s):
    m = pl.program_id(0)

    @pl.when(m < NB)
    def _():
        w1s[m] = w1_ref[0].astype(jnp.bfloat16)
        w2s[m] = (w2_ref[0] * 0.5).astype(jnp.bfloat16)
        b1s[m] = b1_ref[0, 0]

    @pl.when(m >= NB)
    def _():
        xb = x_ref[...].astype(jnp.bfloat16)
        acc = None
        for i in range(NB):
            h = jnp.dot(xb, w1s[i], preferred_element_type=jnp.float32)
            g = _half_gelu(h + b1s[i])
            p = jnp.dot(g.astype(jnp.bfloat16), w2s[i],
                        preferred_element_type=jnp.float32)
            acc = p if acc is None else acc + p
        o_ref[...] = acc + b2_ref[0]


def kernel(hidden_states, W1, b1, W2, b2, col):
    T, D_MODEL = hidden_states.shape
    E, _, D_FF = W1.shape
    col_arr = jnp.asarray(col, jnp.int32).reshape((1,))
    b1r = b1.reshape(E, NB, 1, CB)
    b2r = b2.reshape(E, 1, D_MODEL)
    nm = T // TM

    grid_spec = pltpu.PrefetchScalarGridSpec(
        num_scalar_prefetch=1,
        grid=(NB + nm,),
        in_specs=[
            pl.BlockSpec((TM, D_MODEL),
                         lambda m, c: (jnp.maximum(m - NB, 0), 0)),
            pl.BlockSpec((1, D_MODEL, CB),
                         lambda m, c: (c[0], 0, jnp.minimum(m, NB - 1))),
            pl.BlockSpec((1, 1, 1, CB),
                         lambda m, c: (c[0], jnp.minimum(m, NB - 1), 0, 0)),
            pl.BlockSpec((1, CB, D_MODEL),
                         lambda m, c: (c[0], jnp.minimum(m, NB - 1), 0)),
            pl.BlockSpec((1, 1, D_MODEL), lambda m, c: (c[0], 0, 0)),
        ],
        out_specs=pl.BlockSpec((TM, D_MODEL),
                               lambda m, c: (jnp.maximum(m - NB, 0), 0)),
        scratch_shapes=[
            pltpu.VMEM((NB, D_MODEL, CB), jnp.bfloat16),
            pltpu.VMEM((NB, CB, D_MODEL), jnp.bfloat16),
            pltpu.VMEM((NB, 1, CB), jnp.float32),
        ],
    )
    return pl.pallas_call(
        _mlp_kernel,
        grid_spec=grid_spec,
        out_shape=jax.ShapeDtypeStruct((T, D_MODEL), jnp.float32),
        compiler_params=pltpu.CompilerParams(
            dimension_semantics=("arbitrary",),
        ),
    )(col_arr, hidden_states, W1, b1r, W2, b2r)


# speedup vs baseline: 1.2833x; 1.0259x over previous
import jax
import jax.numpy as jnp
from jax.experimental import pallas as pl
from jax.experimental.pallas import tpu as pltpu

TM = 1024
NB = 4
CB = 1024


def _half_gelu(h):
    c0 = 0.7978845608028654
    c1 = 0.044715 * c0
    u = h * (c0 + (h * h) * c1)
    t = jnp.tanh(u)
    return h + h * t


def _mlp_kernel(col_ref, x_ref, w1_ref, b1_ref, w2_ref, b2_ref, o_ref,
                w1s, w2s, b1s):
    m = pl.program_id(0)

    @pl.when(m < NB)
    def _():
        w1s[m] = w1_ref[0].astype(jnp.bfloat16)
        w2s[m] = (w2_ref[0] * 0.5).astype(jnp.bfloat16)
        b1s[m] = b1_ref[0, 0]

    @pl.when(m >= NB)
    def _():
        xb = x_ref[...].astype(jnp.bfloat16)
        acc = None
        for i in range(NB):
            h = jnp.dot(xb, w1s[i], preferred_element_type=jnp.float32)
            g = _half_gelu(h + b1s[i])
            p = jnp.dot(g.astype(jnp.bfloat16), w2s[i],
                        preferred_element_type=jnp.float32)
            acc = p if acc is None else acc + p
        o_ref[...] = acc + b2_ref[0]


def kernel(hidden_states, W1, b1, W2, b2, col):
    T, D_MODEL = hidden_states.shape
    E, _, D_FF = W1.shape
    col_arr = jnp.asarray(col, jnp.int32).reshape((1,))
    b1r = b1.reshape(E, NB, 1, CB)
    b2r = b2.reshape(E, 1, D_MODEL)
    nm = T // TM

    grid_spec = pltpu.PrefetchScalarGridSpec(
        num_scalar_prefetch=1,
        grid=(NB + nm,),
        in_specs=[
            pl.BlockSpec((TM, D_MODEL),
                         lambda m, c: (jnp.maximum(m - NB, 0), 0)),
            pl.BlockSpec((1, D_MODEL, CB),
                         lambda m, c: (c[0], 0, jnp.minimum(m, NB - 1))),
            pl.BlockSpec((1, 1, 1, CB),
                         lambda m, c: (c[0], jnp.minimum(m, NB - 1), 0, 0)),
            pl.BlockSpec((1, CB, D_MODEL),
                         lambda m, c: (c[0], jnp.minimum(m, NB - 1), 0)),
            pl.BlockSpec((1, 1, D_MODEL), lambda m, c: (c[0], 0, 0)),
        ],
        out_specs=pl.BlockSpec((TM, D_MODEL),
                               lambda m, c: (jnp.maximum(m - NB, 0), 0)),
        scratch_shapes=[
            pltpu.VMEM((NB, D_MODEL, CB), jnp.bfloat16),
            pltpu.VMEM((NB, CB, D_MODEL), jnp.bfloat16),
            pltpu.VMEM((NB, 1, CB), jnp.float32),
        ],
    )
    return pl.pallas_call(
        _mlp_kernel,
        grid_spec=grid_spec,
        out_shape=jax.ShapeDtypeStruct((T, D_MODEL), jnp.float32),
        compiler_params=pltpu.CompilerParams(
            dimension_semantics=("arbitrary",),
        ),
    )(col_arr, hidden_states, W1, b1r, W2, b2r)


# speedup vs baseline: 1.3169x; 1.0261x over previous
import jax
import jax.numpy as jnp
from jax.experimental import pallas as pl
from jax.experimental.pallas import tpu as pltpu

TM = 1024
NB = 4
CB = 1024


def _half_gelu(h):
    c0 = 0.7978845608028654
    c1 = 0.044715 * c0
    u = h * (c0 + (h * h) * c1)
    t = jnp.tanh(u)
    return h + h * t


def _mlp_kernel(col_ref, x_ref, w1_ref, b1_ref, w2_ref, b2_ref, o_ref,
                w1s, w2s, b1s):
    m = pl.program_id(0)

    @pl.when(m < NB)
    def _():
        w1c = w1_ref[0].astype(jnp.bfloat16)
        w2c = (w2_ref[0] * 0.5).astype(jnp.bfloat16)
        b1c = b1_ref[0, 0]
        w1s[m] = w1c
        w2s[m] = w2c
        b1s[m] = b1c

        xb = x_ref[...].astype(jnp.bfloat16)
        h = jnp.dot(xb, w1c, preferred_element_type=jnp.float32)
        g = _half_gelu(h + b1c)
        p = jnp.dot(g.astype(jnp.bfloat16), w2c,
                    preferred_element_type=jnp.float32)

        @pl.when(m == 0)
        def _():
            o_ref[...] = p + b2_ref[0]

        @pl.when(m > 0)
        def _():
            o_ref[...] += p

    @pl.when(m >= NB)
    def _():
        xb = x_ref[...].astype(jnp.bfloat16)
        parts = []
        for i in range(NB):
            h = jnp.dot(xb, w1s[i], preferred_element_type=jnp.float32)
            g = _half_gelu(h + b1s[i])
            parts.append(g.astype(jnp.bfloat16))
        gb = jnp.concatenate(parts, axis=1)
        acc = None
        for i in range(NB):
            p = jnp.dot(gb[:, i * CB:(i + 1) * CB], w2s[i],
                        preferred_element_type=jnp.float32)
            acc = p if acc is None else acc + p
        o_ref[...] = acc + b2_ref[0]


def kernel(hidden_states, W1, b1, W2, b2, col):
    T, D_MODEL = hidden_states.shape
    E, _, D_FF = W1.shape
    col_arr = jnp.asarray(col, jnp.int32).reshape((1,))
    b1r = b1.reshape(E, NB, 1, CB)
    b2r = b2.reshape(E, 1, D_MODEL)
    nm = T // TM

    grid_spec = pltpu.PrefetchScalarGridSpec(
        num_scalar_prefetch=1,
        grid=(NB + nm - 1,),
        in_specs=[
            pl.BlockSpec((TM, D_MODEL),
                         lambda m, c: (jnp.maximum(m - NB + 1, 0), 0)),
            pl.BlockSpec((1, D_MODEL, CB),
                         lambda m, c: (c[0], 0, jnp.minimum(m, NB - 1))),
            pl.BlockSpec((1, 1, 1, CB),
                         lambda m, c: (c[0], jnp.minimum(m, NB - 1), 0, 0)),
            pl.BlockSpec((1, CB, D_MODEL),
                         lambda m, c: (c[0], jnp.minimum(m, NB - 1), 0)),
            pl.BlockSpec((1, 1, D_MODEL), lambda m, c: (c[0], 0, 0)),
        ],
        out_specs=pl.BlockSpec((TM, D_MODEL),
                               lambda m, c: (jnp.maximum(m - NB + 1, 0), 0)),
        scratch_shapes=[
            pltpu.VMEM((NB, D_MODEL, CB), jnp.bfloat16),
            pltpu.VMEM((NB, CB, D_MODEL), jnp.bfloat16),
            pltpu.VMEM((NB, 1, CB), jnp.float32),
        ],
    )
    return pl.pallas_call(
        _mlp_kernel,
        grid_spec=grid_spec,
        out_shape=jax.ShapeDtypeStruct((T, D_MODEL), jnp.float32),
        compiler_params=pltpu.CompilerParams(
            dimension_semantics=("arbitrary",),
        ),
    )(col_arr, hidden_states, W1, b1r, W2, b2r)
